# baseline (speedup 1.0000x reference)
import functools

import jax
import jax.numpy as jnp
from jax import lax
from jax.experimental import pallas as pl
from jax.experimental.pallas import tpu as pltpu
from jax.experimental.pallas import tpu_sc as plsc

VOCAB = 100000
EMB_DIM = 128
N_LABELS = 50
BATCH = 4096
SENT_LEN = 50

NC, NS = 2, 16
NW = NC * NS
PER_W = BATCH // NW
D = 8
K = 4
T_END = ((PER_W + K + D - 1) // D) * D

LBLK = 128


@functools.partial(
    pl.kernel,
    out_type=jax.ShapeDtypeStruct((BATCH, SENT_LEN, EMB_DIM), jnp.float32),
    mesh=plsc.VectorSubcoreMesh(core_axis_name="c", subcore_axis_name="s"),
    scratch_types=(
        [pltpu.VMEM((PER_W, SENT_LEN), jnp.int32)]
        + [pltpu.VMEM((SENT_LEN, EMB_DIM), jnp.float32)] * D
        + [pltpu.SemaphoreType.DMA] * (2 * D)
    ),
)
def _word_lookup_sc(idx_hbm, tab_hbm, out_hbm, idx_v, *bufs_and_sems):
    bufs = bufs_and_sems[:D]
    gsems = bufs_and_sems[D:2 * D]
    ssems = bufs_and_sems[2 * D:]
    wid = lax.axis_index("s") * NC + lax.axis_index("c")
    base = wid * PER_W
    pltpu.sync_copy(idx_hbm.at[pl.ds(base, PER_W)], idx_v)

    @pl.loop(0, T_END, step=D)
    def _step(jj):
        for bi in range(D):
            t = jj + bi
            bd = (bi - K) % D

            @pl.when(jnp.logical_and(t >= D, t < PER_W))
            def _buffer_free():
                pltpu.make_async_copy(bufs[bi], out_hbm.at[0], ssems[bi]).wait()

            @pl.when(t < PER_W)
            def _fire_gather():
                pltpu.async_copy(tab_hbm.at[idx_v.at[t]], bufs[bi], gsems[bi])

            @pl.when(jnp.logical_and(t >= K, t < PER_W + K))
            def _drain():
                c = t - K
                pltpu.make_async_copy(
                    tab_hbm.at[idx_v.at[c]], bufs[bd], gsems[bd]).wait()
                pltpu.async_copy(bufs[bd], out_hbm.at[base + c], ssems[bd])

    for b in range(D):
        pltpu.make_async_copy(bufs[b], out_hbm.at[0], ssems[b]).wait()


def _label_lookup_body(idx_ref, tab_ref, out_ref):
    idx = idx_ref[...]
    iota = lax.broadcasted_iota(jnp.int32, (LBLK, N_LABELS, N_LABELS), 2)
    onehot = (idx[:, :, None] == iota).astype(jnp.float32)
    res = jnp.dot(
        onehot.reshape(LBLK * N_LABELS, N_LABELS), tab_ref[...],
        preferred_element_type=jnp.float32,
        precision=lax.Precision.HIGHEST)
    out_ref[...] = res.reshape(LBLK, N_LABELS, EMB_DIM)


_label_lookup_tc = pl.pallas_call(
    _label_lookup_body,
    grid=(BATCH // LBLK,),
    in_specs=[
        pl.BlockSpec((LBLK, N_LABELS), lambda i: (i, 0)),
        pl.BlockSpec((N_LABELS, EMB_DIM), lambda i: (0, 0)),
    ],
    out_specs=pl.BlockSpec((LBLK, N_LABELS, EMB_DIM), lambda i: (i, 0, 0)),
    out_shape=jax.ShapeDtypeStruct((BATCH, N_LABELS, EMB_DIM), jnp.float32),
)


def kernel(word_inputs, input_label_seq_tensor, word_table, label_table):
    widx = word_inputs.astype(jnp.int32)
    lidx = input_label_seq_tensor.astype(jnp.int32)
    word_represent = _word_lookup_sc(widx, word_table)
    label_embs = _label_lookup_tc(lidx, label_table)
    return (word_represent, label_embs)


# speedup vs baseline: 2.4176x; 2.4176x over previous
import functools

import jax
import jax.numpy as jnp
from jax import lax
from jax.experimental import pallas as pl
from jax.experimental.pallas import tpu as pltpu
from jax.experimental.pallas import tpu_sc as plsc

VOCAB = 100000
EMB_DIM = 128
N_LABELS = 50
BATCH = 4096
SENT_LEN = 50

NC, NS = 2, 16
NW = NC * NS
PER_W = BATCH // NW
D = 4
K = 2
T_END = ((SENT_LEN + K + D - 1) // D) * D

LBLK = 256


@functools.partial(
    pl.kernel,
    out_type=jax.ShapeDtypeStruct((SENT_LEN * BATCH, EMB_DIM), jnp.float32),
    mesh=plsc.VectorSubcoreMesh(core_axis_name="c", subcore_axis_name="s"),
    scratch_types=(
        [pltpu.VMEM((SENT_LEN, PER_W), jnp.int32)]
        + [pltpu.VMEM((PER_W, EMB_DIM), jnp.float32)] * D
        + [pltpu.SemaphoreType.DMA] * (2 * D)
    ),
)
def _word_lookup_sc(idxt_hbm, tab_hbm, out_hbm, idx_v, *bufs_and_sems):
    bufs = bufs_and_sems[:D]
    gsems = bufs_and_sems[D:2 * D]
    ssems = bufs_and_sems[2 * D:]
    wid = lax.axis_index("s") * NC + lax.axis_index("c")
    base = wid * PER_W
    pltpu.sync_copy(idxt_hbm.at[:, pl.ds(base, PER_W)], idx_v)

    @pl.loop(0, T_END, step=D)
    def _step(jj):
        for bi in range(D):
            t = jj + bi
            bd = (bi - K) % D

            @pl.when(jnp.logical_and(t >= D, t < SENT_LEN))
            def _buffer_free():
                pltpu.make_async_copy(
                    bufs[bi], out_hbm.at[pl.ds(0, PER_W)], ssems[bi]).wait()

            @pl.when(t < SENT_LEN)
            def _fire_gather():
                pltpu.async_copy(tab_hbm.at[idx_v.at[t]], bufs[bi], gsems[bi])

            @pl.when(jnp.logical_and(t >= K, t < SENT_LEN + K))
            def _drain():
                c = t - K
                pltpu.make_async_copy(
                    tab_hbm.at[idx_v.at[c]], bufs[bd], gsems[bd]).wait()
                row = pl.multiple_of(c * BATCH + base, PER_W)
                pltpu.async_copy(
                    bufs[bd], out_hbm.at[pl.ds(row, PER_W)], ssems[bd])

    for b in range(D):
        pltpu.make_async_copy(
            bufs[b], out_hbm.at[pl.ds(0, PER_W)], ssems[b]).wait()


def _label_lookup_body(idxt_ref, tab_ref, out_ref):
    idxt = idxt_ref[...]
    iota = lax.broadcasted_iota(jnp.int32, (N_LABELS, LBLK, N_LABELS), 2)
    onehot = (idxt[:, :, None] == iota).astype(jnp.float32)
    res = jnp.dot(
        onehot.reshape(N_LABELS * LBLK, N_LABELS), tab_ref[...],
        preferred_element_type=jnp.float32,
        precision=lax.Precision.HIGHEST)
    out_ref[...] = res.reshape(N_LABELS, LBLK, EMB_DIM)


_label_lookup_tc = pl.pallas_call(
    _label_lookup_body,
    grid=(BATCH // LBLK,),
    in_specs=[
        pl.BlockSpec((N_LABELS, LBLK), lambda i: (0, i)),
        pl.BlockSpec((N_LABELS, EMB_DIM), lambda i: (0, 0)),
    ],
    out_specs=pl.BlockSpec((N_LABELS, LBLK, EMB_DIM), lambda i: (0, i, 0)),
    out_shape=jax.ShapeDtypeStruct((N_LABELS, BATCH, EMB_DIM), jnp.float32),
)


def kernel(word_inputs, input_label_seq_tensor, word_table, label_table):
    widx_t = word_inputs.astype(jnp.int32).T
    lidx_t = input_label_seq_tensor.astype(jnp.int32).T
    wout = _word_lookup_sc(widx_t, word_table)
    lout = _label_lookup_tc(lidx_t, label_table)
    word_represent = wout.reshape(SENT_LEN, BATCH, EMB_DIM).transpose(1, 0, 2)
    label_embs = lout.transpose(1, 0, 2)
    return (word_represent, label_embs)


# speedup vs baseline: 2.4553x; 1.0156x over previous
import functools

import jax
import jax.numpy as jnp
from jax import lax
from jax.experimental import pallas as pl
from jax.experimental.pallas import tpu as pltpu
from jax.experimental.pallas import tpu_sc as plsc

VOCAB = 100000
EMB_DIM = 128
N_LABELS = 50
BATCH = 4096
SENT_LEN = 50

NC, NS = 2, 16
NW = NC * NS
PER_W = BATCH // NW
D = 6
K = 3
T_END = ((SENT_LEN + K + D - 1) // D) * D

LBLK = 256


@functools.partial(
    pl.kernel,
    out_type=jax.ShapeDtypeStruct((SENT_LEN * BATCH, EMB_DIM), jnp.float32),
    mesh=plsc.VectorSubcoreMesh(core_axis_name="c", subcore_axis_name="s"),
    scratch_types=(
        [pltpu.VMEM((SENT_LEN, PER_W), jnp.int32)]
        + [pltpu.VMEM((PER_W, EMB_DIM), jnp.float32)] * D
        + [pltpu.SemaphoreType.DMA] * (2 * D)
    ),
)
def _word_lookup_sc(idxt_hbm, tab_hbm, out_hbm, idx_v, *bufs_and_sems):
    bufs = bufs_and_sems[:D]
    gsems = bufs_and_sems[D:2 * D]
    ssems = bufs_and_sems[2 * D:]
    wid = lax.axis_index("s") * NC + lax.axis_index("c")
    base = wid * PER_W
    pltpu.sync_copy(idxt_hbm.at[:, pl.ds(base, PER_W)], idx_v)

    @pl.loop(0, T_END, step=D)
    def _step(jj):
        for bi in range(D):
            t = jj + bi
            bd = (bi - K) % D

            @pl.when(jnp.logical_and(t >= D, t < SENT_LEN))
            def _buffer_free():
                pltpu.make_async_copy(
                    bufs[bi], out_hbm.at[pl.ds(0, PER_W)], ssems[bi]).wait()

            @pl.when(t < SENT_LEN)
            def _fire_gather():
                pltpu.async_copy(tab_hbm.at[idx_v.at[t]], bufs[bi], gsems[bi])

            @pl.when(jnp.logical_and(t >= K, t < SENT_LEN + K))
            def _drain():
                c = t - K
                pltpu.make_async_copy(
                    tab_hbm.at[idx_v.at[c]], bufs[bd], gsems[bd]).wait()
                row = pl.multiple_of(c * BATCH + base, PER_W)
                pltpu.async_copy(
                    bufs[bd], out_hbm.at[pl.ds(row, PER_W)], ssems[bd])

    for b in range(D):
        pltpu.make_async_copy(
            bufs[b], out_hbm.at[pl.ds(0, PER_W)], ssems[b]).wait()


def _label_lookup_body(idxt_ref, tab_ref, out_ref):
    idxt = idxt_ref[...]
    iota = lax.broadcasted_iota(jnp.int32, (N_LABELS, LBLK, N_LABELS), 2)
    onehot = (idxt[:, :, None] == iota).astype(jnp.float32)
    res = jnp.dot(
        onehot.reshape(N_LABELS * LBLK, N_LABELS), tab_ref[...],
        preferred_element_type=jnp.float32)
    out_ref[...] = res.reshape(N_LABELS, LBLK, EMB_DIM)


_label_lookup_tc = pl.pallas_call(
    _label_lookup_body,
    grid=(BATCH // LBLK,),
    in_specs=[
        pl.BlockSpec((N_LABELS, LBLK), lambda i: (0, i)),
        pl.BlockSpec((N_LABELS, EMB_DIM), lambda i: (0, 0)),
    ],
    out_specs=pl.BlockSpec((N_LABELS, LBLK, EMB_DIM), lambda i: (0, i, 0)),
    out_shape=jax.ShapeDtypeStruct((N_LABELS, BATCH, EMB_DIM), jnp.float32),
)


def kernel(word_inputs, input_label_seq_tensor, word_table, label_table):
    widx_t = word_inputs.astype(jnp.int32).T
    lidx_t = input_label_seq_tensor.astype(jnp.int32).T
    wout = _word_lookup_sc(widx_t, word_table)
    lout = _label_lookup_tc(lidx_t, label_table)
    word_represent = wout.reshape(SENT_LEN, BATCH, EMB_DIM).transpose(1, 0, 2)
    label_embs = lout.transpose(1, 0, 2)
    return (word_represent, label_embs)
